# baseline (speedup 1.0000x reference)
import functools

import jax
import jax.numpy as jnp
from jax import lax
from jax.experimental import pallas as pl
from jax.experimental.pallas import tpu as pltpu
from jax.experimental.pallas import tpu_sc as plsc

_EPS = 1e-12


@functools.lru_cache(maxsize=None)
def _make_sc_gather(N: int, V: int, D: int):
    info = plsc.get_sparse_core_info()
    NC, NS = info.num_cores, info.num_subcores
    NW = NC * NS
    assert N % NW == 0
    b_per_w = N // NW
    CH = 640
    while b_per_w % CH:
        CH //= 2
    n_ch = b_per_w // CH
    mesh = plsc.VectorSubcoreMesh(core_axis_name="c", subcore_axis_name="s")

    @functools.partial(
        pl.kernel,
        mesh=mesh,
        out_type=jax.ShapeDtypeStruct((N, D), jnp.float32),
        scratch_types=[
            pltpu.VMEM((CH,), jnp.int32),
            pltpu.VMEM((CH, D), jnp.float32),
            pltpu.SemaphoreType.DMA,
        ],
    )
    def gather_kernel(idx_hbm, table_hbm, out_hbm, idx_v, rows_v, sem):
        wid = lax.axis_index("s") * NC + lax.axis_index("c")
        base = wid * b_per_w

        def body(i, carry):
            off = base + i * CH
            pltpu.sync_copy(idx_hbm.at[pl.ds(off, CH)], idx_v)
            pltpu.async_copy(table_hbm.at[idx_v], rows_v, sem).wait()
            pltpu.sync_copy(rows_v, out_hbm.at[pl.ds(off, CH)])
            return carry

        lax.fori_loop(0, n_ch, body, 0)

    return gather_kernel


def _proj_ln_body(g_ref, w_ref, gamma_ref, beta_ref, _full_ref, o_ref):
    g = g_ref[...]
    w = w_ref[...]
    h = jnp.dot(g, w, preferred_element_type=jnp.float32)
    mean = jnp.mean(h, axis=-1, keepdims=True)
    c = h - mean
    var = jnp.mean(c * c, axis=-1, keepdims=True)
    o_ref[...] = (c * lax.rsqrt(var + _EPS)) * gamma_ref[...] + beta_ref[...]


def _proj_ln_slice(g_s, W, gamma, beta, out_prev, base_blk, T):
    NS, D = g_s.shape
    N, H = out_prev.shape
    return pl.pallas_call(
        _proj_ln_body,
        grid=(NS // T,),
        in_specs=[
            pl.BlockSpec((T, D), lambda i: (i, 0)),
            pl.BlockSpec((D, H), lambda i: (0, 0)),
            pl.BlockSpec((1, H), lambda i: (0, 0)),
            pl.BlockSpec((1, H), lambda i: (0, 0)),
            pl.BlockSpec(memory_space=pl.ANY),
        ],
        out_specs=pl.BlockSpec((T, H), lambda i: (i + base_blk, 0)),
        out_shape=jax.ShapeDtypeStruct((N, H), jnp.float32),
        input_output_aliases={4: 0},
    )(g_s, W, gamma.reshape(1, H), beta.reshape(1, H), out_prev)


def kernel(entity_ids, table, W, gamma, beta):
    B, L = entity_ids.shape
    N = B * L
    V, D = table.shape
    H = W.shape[1]
    idx = entity_ids.T.reshape(N).astype(jnp.int32)
    S = 5
    T = 4096
    NS = N // S
    gather = _make_sc_gather(NS, V, D)
    g_slices = [gather(lax.slice_in_dim(idx, s * NS, (s + 1) * NS), table)
                for s in range(S)]
    out = jnp.zeros((0,))
    for s in range(S):
        if s == 0:
            out = pl.pallas_call(
                _proj_ln_body,
                grid=(NS // T,),
                in_specs=[
                    pl.BlockSpec((T, D), lambda i: (i, 0)),
                    pl.BlockSpec((D, H), lambda i: (0, 0)),
                    pl.BlockSpec((1, H), lambda i: (0, 0)),
                    pl.BlockSpec((1, H), lambda i: (0, 0)),
                    pl.BlockSpec(memory_space=pl.ANY),
                ],
                out_specs=pl.BlockSpec((T, H), lambda i: (i, 0)),
                out_shape=jax.ShapeDtypeStruct((N, H), jnp.float32),
            )(g_slices[0], W, gamma.reshape(1, H), beta.reshape(1, H),
              g_slices[0])
        else:
            out = _proj_ln_slice(g_slices[s], W, gamma, beta, out,
                                 base_blk=s * (NS // T), T=T)
    return out.reshape(L, B, H).transpose(1, 0, 2)


# speedup vs baseline: 1.0001x; 1.0001x over previous
import functools

import jax
import jax.numpy as jnp
from jax import lax
from jax.experimental import pallas as pl
from jax.experimental.pallas import tpu as pltpu
from jax.experimental.pallas import tpu_sc as plsc

_EPS = 1e-12


@functools.lru_cache(maxsize=None)
def _make_sc_gather(N: int, V: int, D: int):
    info = plsc.get_sparse_core_info()
    NC, NS = info.num_cores, info.num_subcores
    NW = NC * NS
    assert N % NW == 0
    b_per_w = N // NW
    CH = 8
    for c in range(8, min(640, b_per_w) + 1, 8):
        if b_per_w % c == 0:
            CH = c
    n_ch = b_per_w // CH
    mesh = plsc.VectorSubcoreMesh(core_axis_name="c", subcore_axis_name="s")

    @functools.partial(
        pl.kernel,
        mesh=mesh,
        out_type=jax.ShapeDtypeStruct((N, D), jnp.float32),
        scratch_types=[
            pltpu.VMEM((CH,), jnp.int32),
            pltpu.VMEM((CH, D), jnp.float32),
            pltpu.SemaphoreType.DMA,
        ],
    )
    def gather_kernel(idx_hbm, table_hbm, out_hbm, idx_v, rows_v, sem):
        wid = lax.axis_index("s") * NC + lax.axis_index("c")
        base = wid * b_per_w

        def body(i, carry):
            off = base + i * CH
            pltpu.sync_copy(idx_hbm.at[pl.ds(off, CH)], idx_v)
            pltpu.async_copy(table_hbm.at[idx_v], rows_v, sem).wait()
            pltpu.sync_copy(rows_v, out_hbm.at[pl.ds(off, CH)])
            return carry

        lax.fori_loop(0, n_ch, body, 0)

    return gather_kernel


def _proj_ln_body(g_ref, w_ref, gamma_ref, beta_ref, _full_ref, o_ref):
    g = g_ref[...]
    w = w_ref[...]
    h = jnp.dot(g, w, preferred_element_type=jnp.float32)
    mean = jnp.mean(h, axis=-1, keepdims=True)
    c = h - mean
    var = jnp.mean(c * c, axis=-1, keepdims=True)
    o_ref[...] = (c * lax.rsqrt(var + _EPS)) * gamma_ref[...] + beta_ref[...]


def _proj_ln_slice(g_s, W, gamma, beta, out_prev, base_blk, T):
    NS, D = g_s.shape
    N, H = out_prev.shape
    return pl.pallas_call(
        _proj_ln_body,
        grid=(NS // T,),
        in_specs=[
            pl.BlockSpec((T, D), lambda i: (i, 0)),
            pl.BlockSpec((D, H), lambda i: (0, 0)),
            pl.BlockSpec((1, H), lambda i: (0, 0)),
            pl.BlockSpec((1, H), lambda i: (0, 0)),
            pl.BlockSpec(memory_space=pl.ANY),
        ],
        out_specs=pl.BlockSpec((T, H), lambda i: (i + base_blk, 0)),
        out_shape=jax.ShapeDtypeStruct((N, H), jnp.float32),
        input_output_aliases={4: 0},
    )(g_s, W, gamma.reshape(1, H), beta.reshape(1, H), out_prev)


def kernel(entity_ids, table, W, gamma, beta):
    B, L = entity_ids.shape
    N = B * L
    V, D = table.shape
    H = W.shape[1]
    idx = entity_ids.T.reshape(N).astype(jnp.int32)
    T = 4096
    slice_blks = [4, 12, 12, 11, 11]
    S = len(slice_blks)
    offs, acc = [], 0
    for nb in slice_blks:
        offs.append(acc)
        acc += nb * T
    g_slices = [
        _make_sc_gather(nb * T, V, D)(
            lax.slice_in_dim(idx, off, off + nb * T), table)
        for nb, off in zip(slice_blks, offs)
    ]
    out = jnp.zeros((0,))
    for s in range(S):
        if s == 0:
            out = pl.pallas_call(
                _proj_ln_body,
                grid=(slice_blks[0],),
                in_specs=[
                    pl.BlockSpec((T, D), lambda i: (i, 0)),
                    pl.BlockSpec((D, H), lambda i: (0, 0)),
                    pl.BlockSpec((1, H), lambda i: (0, 0)),
                    pl.BlockSpec((1, H), lambda i: (0, 0)),
                    pl.BlockSpec(memory_space=pl.ANY),
                ],
                out_specs=pl.BlockSpec((T, H), lambda i: (i, 0)),
                out_shape=jax.ShapeDtypeStruct((N, H), jnp.float32),
            )(g_slices[0], W, gamma.reshape(1, H), beta.reshape(1, H),
              g_slices[0])
        else:
            out = _proj_ln_slice(g_slices[s], W, gamma, beta, out,
                                 base_blk=offs[s] // T, T=T)
    return out.reshape(L, B, H).transpose(1, 0, 2)


# speedup vs baseline: 1.0237x; 1.0236x over previous
import functools

import jax
import jax.numpy as jnp
from jax import lax
from jax.experimental import pallas as pl
from jax.experimental.pallas import tpu as pltpu
from jax.experimental.pallas import tpu_sc as plsc

_EPS = 1e-12


@functools.lru_cache(maxsize=None)
def _make_sc_gather(N: int, V: int, D: int):
    info = plsc.get_sparse_core_info()
    NC, NS = info.num_cores, info.num_subcores
    NW = NC * NS
    assert N % NW == 0
    b_per_w = N // NW
    CH = 8
    for c in range(8, min(640, b_per_w) + 1, 8):
        if b_per_w % c == 0:
            CH = c
    n_ch = b_per_w // CH
    mesh = plsc.VectorSubcoreMesh(core_axis_name="c", subcore_axis_name="s")

    @functools.partial(
        pl.kernel,
        mesh=mesh,
        out_type=jax.ShapeDtypeStruct((N, D), jnp.float32),
        scratch_types=[
            pltpu.VMEM((CH,), jnp.int32),
            pltpu.VMEM((CH, D), jnp.float32),
            pltpu.SemaphoreType.DMA,
        ],
    )
    def gather_kernel(idx_hbm, table_hbm, out_hbm, idx_v, rows_v, sem):
        wid = lax.axis_index("s") * NC + lax.axis_index("c")
        base = wid * b_per_w

        def body(i, carry):
            off = base + i * CH
            pltpu.sync_copy(idx_hbm.at[pl.ds(off, CH)], idx_v)
            pltpu.async_copy(table_hbm.at[idx_v], rows_v, sem).wait()
            pltpu.sync_copy(rows_v, out_hbm.at[pl.ds(off, CH)])
            return carry

        lax.fori_loop(0, n_ch, body, 0)

    return gather_kernel


def _proj_ln_body(g_ref, w_ref, gamma_ref, beta_ref, _full_ref, o_ref):
    g = g_ref[...]
    w = w_ref[...]
    h = jnp.dot(g, w, preferred_element_type=jnp.float32)
    mean = jnp.mean(h, axis=-1, keepdims=True)
    c = h - mean
    var = jnp.mean(c * c, axis=-1, keepdims=True)
    o_ref[...] = (c * lax.rsqrt(var + _EPS)) * gamma_ref[...] + beta_ref[...]


def _proj_ln_slice(g_s, W, gamma, beta, out_prev, base_blk, T):
    NS, D = g_s.shape
    N, H = out_prev.shape
    return pl.pallas_call(
        _proj_ln_body,
        grid=(NS // T,),
        in_specs=[
            pl.BlockSpec((T, D), lambda i: (i, 0)),
            pl.BlockSpec((D, H), lambda i: (0, 0)),
            pl.BlockSpec((1, H), lambda i: (0, 0)),
            pl.BlockSpec((1, H), lambda i: (0, 0)),
            pl.BlockSpec(memory_space=pl.ANY),
        ],
        out_specs=pl.BlockSpec((T, H), lambda i: (i + base_blk, 0)),
        out_shape=jax.ShapeDtypeStruct((N, H), jnp.float32),
        input_output_aliases={4: 0},
    )(g_s, W, gamma.reshape(1, H), beta.reshape(1, H), out_prev)


def kernel(entity_ids, table, W, gamma, beta):
    B, L = entity_ids.shape
    N = B * L
    V, D = table.shape
    H = W.shape[1]
    idx = entity_ids.T.reshape(N).astype(jnp.int32)
    T = 4096
    slice_blks = [8, 42]
    S = len(slice_blks)
    offs, acc = [], 0
    for nb in slice_blks:
        offs.append(acc)
        acc += nb * T
    g_slices = [
        _make_sc_gather(nb * T, V, D)(
            lax.slice_in_dim(idx, off, off + nb * T), table)
        for nb, off in zip(slice_blks, offs)
    ]
    out = jnp.zeros((0,))
    for s in range(S):
        if s == 0:
            out = pl.pallas_call(
                _proj_ln_body,
                grid=(slice_blks[0],),
                in_specs=[
                    pl.BlockSpec((T, D), lambda i: (i, 0)),
                    pl.BlockSpec((D, H), lambda i: (0, 0)),
                    pl.BlockSpec((1, H), lambda i: (0, 0)),
                    pl.BlockSpec((1, H), lambda i: (0, 0)),
                    pl.BlockSpec(memory_space=pl.ANY),
                ],
                out_specs=pl.BlockSpec((T, H), lambda i: (i, 0)),
                out_shape=jax.ShapeDtypeStruct((N, H), jnp.float32),
            )(g_slices[0], W, gamma.reshape(1, H), beta.reshape(1, H),
              g_slices[0])
        else:
            out = _proj_ln_slice(g_slices[s], W, gamma, beta, out,
                                 base_blk=offs[s] // T, T=T)
    return out.reshape(L, B, H).transpose(1, 0, 2)
